# baseline (speedup 1.0000x reference)
import functools

import jax
import jax.numpy as jnp
from jax import lax
from jax.experimental import pallas as pl
from jax.experimental.pallas import tpu as pltpu
from jax.experimental.pallas import tpu_sc as plsc

VOCAB_ = 100000
H_ = 128
B_ = 4096
C_ = 100
L_ = 20
EPS_ = 1e-05

N_ROWS = C_ * L_
N_ROWS_PAD = 2048
BB_ = 128


def _make_sc_gather():
    info = plsc.get_sparse_core_info()
    nc, ns = info.num_cores, info.num_subcores
    nw = nc * ns
    b_per_w = N_ROWS_PAD // nw
    mesh = plsc.VectorSubcoreMesh(core_axis_name="c", subcore_axis_name="s")

    @functools.partial(
        pl.kernel,
        mesh=mesh,
        out_type=jax.ShapeDtypeStruct((N_ROWS_PAD, H_), jnp.float32),
        scratch_types=[
            pltpu.VMEM((b_per_w,), jnp.int32),
            pltpu.VMEM((b_per_w, H_), jnp.float32),
            pltpu.SemaphoreType.DMA,
        ],
    )
    def gather_kernel(table_hbm, idx_hbm, out_hbm, idx_v, rows_v, sem):
        wid = lax.axis_index("s") * nc + lax.axis_index("c")
        base = wid * b_per_w
        pltpu.sync_copy(idx_hbm.at[pl.ds(base, b_per_w)], idx_v)
        pltpu.async_copy(table_hbm.at[idx_v], rows_v, sem).wait()
        pltpu.sync_copy(rows_v, out_hbm.at[pl.ds(base, b_per_w)])

    return gather_kernel


def _fused_body(xt_ref, g_ref, sm_ref, lnw_ref, lnb_ref, bias_ref, w_ref,
                o_ref, m_ref, p_s, q_s):
    i = pl.program_id(0)

    @pl.when(i == 0)
    def _():
        g = g_ref[...]
        ones_h = jnp.ones((H_, H_), jnp.float32)
        dn = (((1,), (0,)), ((), ()))
        mu = lax.dot_general(g, ones_h, dn,
                             preferred_element_type=jnp.float32) * (1.0 / H_)
        d = g - mu
        var = lax.dot_general(d * d, ones_h, dn,
                              preferred_element_type=jnp.float32) * (1.0 / H_)
        nrm = d / jnp.sqrt(var + EPS_) * lnw_ref[...] + lnb_ref[...]
        sm = sm_ref[...]
        colnum = lax.dot_general(sm, nrm, dn,
                                 preferred_element_type=jnp.float32)
        den = lax.dot_general(sm, jnp.ones((N_ROWS_PAD, H_), jnp.float32), dn,
                              preferred_element_type=jnp.float32)
        col = colnum / den
        dnt = (((1,), (1,)), ((), ()))
        p_s[...] = lax.dot_general(col, w_ref[...], dnt,
                                   preferred_element_type=jnp.float32)
        q_s[...] = lax.dot_general(bias_ref[...], w_ref[...], dnt,
                                   preferred_element_type=jnp.float32)

    xt3 = lax.broadcast_in_dim(xt_ref[...], (C_, BB_, H_), (0, 1))
    p3 = lax.broadcast_in_dim(p_s[...], (C_, BB_, H_), (0, 2))
    q3 = lax.broadcast_in_dim(q_s[...], (C_, BB_, H_), (0, 2))
    o_ref[...] = xt3 * p3 + q3
    m_ref[...] = jnp.ones((C_, BB_), jnp.float32)


def kernel(x_num, num_col_input_ids, num_att_mask, word_emb, ln_w, ln_b,
           num_bias, align_W):
    ids = num_col_input_ids.reshape(-1).astype(jnp.int32)
    ids = jnp.concatenate(
        [ids, jnp.zeros((N_ROWS_PAD - N_ROWS,), jnp.int32)])

    gathered = _make_sc_gather()(word_emb, ids)

    maskf = num_att_mask.astype(jnp.float32).reshape(-1)
    maskf = jnp.concatenate(
        [maskf, jnp.zeros((N_ROWS_PAD - N_ROWS,), jnp.float32)])
    colmap = jnp.arange(N_ROWS_PAD, dtype=jnp.int32) // L_
    sel = (colmap[None, :] == jnp.arange(C_, dtype=jnp.int32)[:, None])
    sm = sel.astype(jnp.float32) * maskf[None, :]

    xt = x_num.T
    xpd, msk = pl.pallas_call(
        _fused_body,
        grid=(B_ // BB_,),
        in_specs=[
            pl.BlockSpec((C_, BB_), lambda i: (0, i)),
            pl.BlockSpec((N_ROWS_PAD, H_), lambda i: (0, 0)),
            pl.BlockSpec((C_, N_ROWS_PAD), lambda i: (0, 0)),
            pl.BlockSpec((1, H_), lambda i: (0, 0)),
            pl.BlockSpec((1, H_), lambda i: (0, 0)),
            pl.BlockSpec((1, H_), lambda i: (0, 0)),
            pl.BlockSpec((H_, H_), lambda i: (0, 0)),
        ],
        out_specs=[
            pl.BlockSpec((C_, BB_, H_), lambda i: (0, i, 0)),
            pl.BlockSpec((C_, BB_), lambda i: (0, i)),
        ],
        out_shape=[
            jax.ShapeDtypeStruct((C_, B_, H_), jnp.float32),
            jax.ShapeDtypeStruct((C_, B_), jnp.float32),
        ],
        scratch_shapes=[
            pltpu.VMEM((C_, H_), jnp.float32),
            pltpu.VMEM((1, H_), jnp.float32),
        ],
    )(xt, gathered, sm, ln_w.reshape(1, H_), ln_b.reshape(1, H_),
      num_bias.reshape(1, H_), align_W)

    out = xpd.transpose(1, 0, 2)
    attention_mask = msk.T
    return (out, attention_mask)


# speedup vs baseline: 1.0527x; 1.0527x over previous
import functools

import jax
import jax.numpy as jnp
from jax import lax
from jax.experimental import pallas as pl
from jax.experimental.pallas import tpu as pltpu
from jax.experimental.pallas import tpu_sc as plsc

VOCAB_ = 100000
H_ = 128
B_ = 4096
C_ = 100
L_ = 20
EPS_ = 1e-05

N_ROWS = C_ * L_
N_ROWS_PAD = 2048
BB_ = 256


def _make_sc_gather():
    info = plsc.get_sparse_core_info()
    nc, ns = info.num_cores, info.num_subcores
    nw = nc * ns
    b_per_w = N_ROWS_PAD // nw
    mesh = plsc.VectorSubcoreMesh(core_axis_name="c", subcore_axis_name="s")

    @functools.partial(
        pl.kernel,
        mesh=mesh,
        out_type=jax.ShapeDtypeStruct((N_ROWS_PAD, H_), jnp.float32),
        scratch_types=[
            pltpu.VMEM((b_per_w,), jnp.int32),
            pltpu.VMEM((b_per_w, H_), jnp.float32),
            pltpu.SemaphoreType.DMA,
        ],
    )
    def gather_kernel(table_hbm, idx_hbm, out_hbm, idx_v, rows_v, sem):
        wid = lax.axis_index("s") * nc + lax.axis_index("c")
        base = wid * b_per_w
        pltpu.sync_copy(idx_hbm.at[pl.ds(base, b_per_w)], idx_v)
        pltpu.async_copy(table_hbm.at[idx_v], rows_v, sem).wait()
        pltpu.sync_copy(rows_v, out_hbm.at[pl.ds(base, b_per_w)])

    return gather_kernel


def _fused_body(xt_ref, g_ref, sm_ref, lnw_ref, lnb_ref, bias_ref, w_ref,
                o_ref, m_ref, p_s, q_s):
    i = pl.program_id(0)

    @pl.when(i == 0)
    def _():
        g = g_ref[...]
        ones_h = jnp.ones((H_, H_), jnp.float32)
        dn = (((1,), (0,)), ((), ()))
        mu = lax.dot_general(g, ones_h, dn,
                             preferred_element_type=jnp.float32) * (1.0 / H_)
        d = g - mu
        var = lax.dot_general(d * d, ones_h, dn,
                              preferred_element_type=jnp.float32) * (1.0 / H_)
        nrm = d / jnp.sqrt(var + EPS_) * lnw_ref[...] + lnb_ref[...]
        sm = sm_ref[...]
        colnum = lax.dot_general(sm, nrm, dn,
                                 preferred_element_type=jnp.float32)
        den = lax.dot_general(sm, jnp.ones((N_ROWS_PAD, H_), jnp.float32), dn,
                              preferred_element_type=jnp.float32)
        col = colnum / den
        dnt = (((1,), (1,)), ((), ()))
        p_s[...] = lax.dot_general(col, w_ref[...], dnt,
                                   preferred_element_type=jnp.float32)
        q_s[...] = lax.dot_general(bias_ref[...], w_ref[...], dnt,
                                   preferred_element_type=jnp.float32)

    xt3 = lax.broadcast_in_dim(xt_ref[...], (C_, BB_, H_), (0, 1))
    p3 = lax.broadcast_in_dim(p_s[...], (C_, BB_, H_), (0, 2))
    q3 = lax.broadcast_in_dim(q_s[...], (C_, BB_, H_), (0, 2))
    o_ref[...] = xt3 * p3 + q3
    m_ref[...] = jnp.ones((C_, BB_), jnp.float32)


def kernel(x_num, num_col_input_ids, num_att_mask, word_emb, ln_w, ln_b,
           num_bias, align_W):
    ids = num_col_input_ids.reshape(-1).astype(jnp.int32)
    ids = jnp.concatenate(
        [ids, jnp.zeros((N_ROWS_PAD - N_ROWS,), jnp.int32)])

    gathered = _make_sc_gather()(word_emb, ids)

    maskf = num_att_mask.astype(jnp.float32).reshape(-1)
    maskf = jnp.concatenate(
        [maskf, jnp.zeros((N_ROWS_PAD - N_ROWS,), jnp.float32)])
    colmap = jnp.arange(N_ROWS_PAD, dtype=jnp.int32) // L_
    sel = (colmap[None, :] == jnp.arange(C_, dtype=jnp.int32)[:, None])
    sm = sel.astype(jnp.float32) * maskf[None, :]

    xt = x_num.T
    xpd, msk = pl.pallas_call(
        _fused_body,
        grid=(B_ // BB_,),
        in_specs=[
            pl.BlockSpec((C_, BB_), lambda i: (0, i)),
            pl.BlockSpec((N_ROWS_PAD, H_), lambda i: (0, 0)),
            pl.BlockSpec((C_, N_ROWS_PAD), lambda i: (0, 0)),
            pl.BlockSpec((1, H_), lambda i: (0, 0)),
            pl.BlockSpec((1, H_), lambda i: (0, 0)),
            pl.BlockSpec((1, H_), lambda i: (0, 0)),
            pl.BlockSpec((H_, H_), lambda i: (0, 0)),
        ],
        out_specs=[
            pl.BlockSpec((C_, BB_, H_), lambda i: (0, i, 0)),
            pl.BlockSpec((C_, BB_), lambda i: (0, i)),
        ],
        out_shape=[
            jax.ShapeDtypeStruct((C_, B_, H_), jnp.float32),
            jax.ShapeDtypeStruct((C_, B_), jnp.float32),
        ],
        scratch_shapes=[
            pltpu.VMEM((C_, H_), jnp.float32),
            pltpu.VMEM((1, H_), jnp.float32),
        ],
    )(xt, gathered, sm, ln_w.reshape(1, H_), ln_b.reshape(1, H_),
      num_bias.reshape(1, H_), align_W)

    out = xpd.transpose(1, 0, 2)
    attention_mask = msk.T
    return (out, attention_mask)


# speedup vs baseline: 1.0541x; 1.0013x over previous
import functools

import jax
import jax.numpy as jnp
from jax import lax
from jax.experimental import pallas as pl
from jax.experimental.pallas import tpu as pltpu
from jax.experimental.pallas import tpu_sc as plsc

VOCAB_ = 100000
H_ = 128
B_ = 4096
C_ = 100
L_ = 20
EPS_ = 1e-05

N_ROWS = C_ * L_
N_ROWS_PAD = 2048
BB_ = 256


def _make_sc_gather():
    info = plsc.get_sparse_core_info()
    nc, ns = info.num_cores, info.num_subcores
    nw = nc * ns
    b_per_w = N_ROWS_PAD // nw
    mesh = plsc.VectorSubcoreMesh(core_axis_name="c", subcore_axis_name="s")

    @functools.partial(
        pl.kernel,
        mesh=mesh,
        out_type=jax.ShapeDtypeStruct((N_ROWS_PAD, H_), jnp.float32),
        scratch_types=[
            pltpu.VMEM((b_per_w,), jnp.int32),
            pltpu.VMEM((b_per_w, H_), jnp.float32),
            pltpu.SemaphoreType.DMA,
        ],
    )
    def gather_kernel(table_hbm, idx_hbm, out_hbm, idx_v, rows_v, sem):
        wid = lax.axis_index("s") * nc + lax.axis_index("c")
        base = wid * b_per_w
        pltpu.sync_copy(idx_hbm.at[pl.ds(base, b_per_w)], idx_v)
        pltpu.async_copy(table_hbm.at[idx_v], rows_v, sem).wait()
        pltpu.sync_copy(rows_v, out_hbm.at[pl.ds(base, b_per_w)])

    return gather_kernel


def _fused_body(xt_ref, g_ref, sm_ref, lnw_ref, lnb_ref, bias_ref, w_ref,
                o_ref, m_ref, p_s, q_s):
    i = pl.program_id(0)

    @pl.when(i == 0)
    def _():
        g = g_ref[...]
        ones_h = jnp.ones((H_, H_), jnp.float32)
        dn = (((1,), (0,)), ((), ()))
        mu = lax.dot_general(g, ones_h, dn,
                             preferred_element_type=jnp.float32) * (1.0 / H_)
        d = g - mu
        var = lax.dot_general(d * d, ones_h, dn,
                              preferred_element_type=jnp.float32) * (1.0 / H_)
        nrm = d / jnp.sqrt(var + EPS_) * lnw_ref[...] + lnb_ref[...]
        sm = sm_ref[...]
        colnum = lax.dot_general(sm, nrm, dn,
                                 preferred_element_type=jnp.float32)
        den = lax.dot_general(sm, jnp.ones((N_ROWS_PAD, H_), jnp.float32), dn,
                              preferred_element_type=jnp.float32)
        col = colnum / den
        dnt = (((1,), (1,)), ((), ()))
        p_s[...] = lax.dot_general(col, w_ref[...], dnt,
                                   preferred_element_type=jnp.float32)
        q_s[...] = lax.dot_general(bias_ref[...], w_ref[...], dnt,
                                   preferred_element_type=jnp.float32)

    xt3 = lax.broadcast_in_dim(xt_ref[...], (C_, BB_, H_), (0, 1))
    p3 = lax.broadcast_in_dim(p_s[...], (C_, BB_, H_), (0, 2))
    q3 = lax.broadcast_in_dim(q_s[...], (C_, BB_, H_), (0, 2))
    o_ref[...] = xt3 * p3 + q3
    m_ref[...] = jnp.ones((C_, BB_), jnp.float32)


def kernel(x_num, num_col_input_ids, num_att_mask, word_emb, ln_w, ln_b,
           num_bias, align_W):
    ids = num_col_input_ids.T.reshape(-1).astype(jnp.int32)
    ids = jnp.concatenate(
        [ids, jnp.zeros((N_ROWS_PAD - N_ROWS,), jnp.int32)])

    gathered = _make_sc_gather()(word_emb, ids)

    maskf = num_att_mask.T.astype(jnp.float32).reshape(-1)
    maskf = jnp.concatenate(
        [maskf, jnp.zeros((N_ROWS_PAD - N_ROWS,), jnp.float32)])
    colmap = jnp.arange(N_ROWS_PAD, dtype=jnp.int32) % C_
    colmap = jnp.where(jnp.arange(N_ROWS_PAD) < N_ROWS, colmap, C_)
    sel = (colmap[None, :] == jnp.arange(C_, dtype=jnp.int32)[:, None])
    sm = sel.astype(jnp.float32) * maskf[None, :]

    xt = x_num.T
    xpd, msk = pl.pallas_call(
        _fused_body,
        grid=(B_ // BB_,),
        in_specs=[
            pl.BlockSpec((C_, BB_), lambda i: (0, i)),
            pl.BlockSpec((N_ROWS_PAD, H_), lambda i: (0, 0)),
            pl.BlockSpec((C_, N_ROWS_PAD), lambda i: (0, 0)),
            pl.BlockSpec((1, H_), lambda i: (0, 0)),
            pl.BlockSpec((1, H_), lambda i: (0, 0)),
            pl.BlockSpec((1, H_), lambda i: (0, 0)),
            pl.BlockSpec((H_, H_), lambda i: (0, 0)),
        ],
        out_specs=[
            pl.BlockSpec((C_, BB_, H_), lambda i: (0, i, 0)),
            pl.BlockSpec((C_, BB_), lambda i: (0, i)),
        ],
        out_shape=[
            jax.ShapeDtypeStruct((C_, B_, H_), jnp.float32),
            jax.ShapeDtypeStruct((C_, B_), jnp.float32),
        ],
        scratch_shapes=[
            pltpu.VMEM((C_, H_), jnp.float32),
            pltpu.VMEM((1, H_), jnp.float32),
        ],
    )(xt, gathered, sm, ln_w.reshape(1, H_), ln_b.reshape(1, H_),
      num_bias.reshape(1, H_), align_W)

    out = xpd.transpose(1, 0, 2)
    attention_mask = msk.T
    return (out, attention_mask)
